# baseline (speedup 1.0000x reference)
import functools

import jax
import jax.numpy as jnp
from jax import lax
from jax.experimental import pallas as pl
from jax.experimental.pallas import tpu as pltpu
from jax.experimental.pallas import tpu_sc as plsc

MAX_DEG = 64
VOCAB = MAX_DEG + 1
D = 128
N_ROWS = 100000
NUM_CORES = 2
NUM_SUBCORES = 16
NW = NUM_CORES * NUM_SUBCORES
CH = 128
NFULL = N_ROWS // CH
TAIL = N_ROWS - NFULL * CH
NCH = NFULL + 1
PAD_N = NCH * CH
CPW = (NCH + NW - 1) // NW


def _prep_body(win_ref, wout_ref, ind_ref, outd_ref, wsum_ref, idx_ref):
    win = win_ref[...]
    wout = wout_ref[...]
    wsum_ref[...] = win[:, None, :] + wout[None, :, :]
    a = jnp.clip(ind_ref[...], 0, MAX_DEG)
    b = jnp.clip(outd_ref[...], 0, MAX_DEG)
    idx_ref[...] = a * VOCAB + b


_mesh = plsc.VectorSubcoreMesh(core_axis_name="c", subcore_axis_name="s")


@functools.partial(
    pl.kernel,
    mesh=_mesh,
    out_type=jax.ShapeDtypeStruct((N_ROWS, D), jnp.float32),
    scratch_types=[
        pltpu.VMEM((CH,), jnp.int32),
        pltpu.VMEM((CH,), jnp.int32),
        pltpu.VMEM((CH,), jnp.int32),
        pltpu.VMEM((CH,), jnp.int32),
        pltpu.VMEM((CH, D), jnp.float32),
        pltpu.VMEM((CH, D), jnp.float32),
        pltpu.VMEM((CH, D), jnp.float32),
        pltpu.VMEM((CH, D), jnp.float32),
        pltpu.SemaphoreType.DMA,
        pltpu.SemaphoreType.DMA,
        pltpu.SemaphoreType.DMA,
        pltpu.SemaphoreType.DMA,
        pltpu.SemaphoreType.DMA,
        pltpu.SemaphoreType.DMA,
        pltpu.SemaphoreType.DMA,
        pltpu.SemaphoreType.DMA,
        pltpu.SemaphoreType.DMA,
    ],
)
def _sc_gather(wsum_hbm, idx_hbm, out_hbm, idx_a, idx_b, idx_c, idx_d,
               rows_a, rows_b, rows_c, rows_d, sem_i, sem_g0, sem_g1, sem_g2,
               sem_g3, sem_w0, sem_w1, sem_w2, sem_w3):
    wid = lax.axis_index("s") * NUM_CORES + lax.axis_index("c")
    idx = (idx_a, idx_b, idx_c, idx_d)
    rows = (rows_a, rows_b, rows_c, rows_d)
    sem_g = (sem_g0, sem_g1, sem_g2, sem_g3)
    sem_w = (sem_w0, sem_w1, sem_w2, sem_w3)
    M = 4

    def wait_gather(c):
        pltpu.make_async_copy(wsum_hbm.at[pl.ds(0, CH)], rows[c % M],
                              sem_g[c % M]).wait()

    def wait_write(c, g):
        pltpu.make_async_copy(rows[c % M], out_hbm.at[pl.ds(g * CH, CH)],
                              sem_w[c % M]).wait()

    pltpu.sync_copy(idx_hbm.at[pl.ds(wid * CH, CH)], idx[0])
    pltpu.async_copy(wsum_hbm.at[idx[0]], rows[0], sem_g[0])
    pltpu.async_copy(idx_hbm.at[pl.ds((NW + wid) * CH, CH)], idx[1], sem_i)
    pltpu.make_async_copy(idx_hbm.at[pl.ds((NW + wid) * CH, CH)], idx[1],
                          sem_i).wait()
    pltpu.async_copy(wsum_hbm.at[idx[1]], rows[1], sem_g[1])
    pltpu.async_copy(idx_hbm.at[pl.ds((2 * NW + wid) * CH, CH)], idx[2],
                     sem_i)

    for c in range(CPW):
        g = c * NW + wid
        b = c % M
        g2 = g + 2 * NW

        if c + 2 < CPW:
            @pl.when(g2 < NCH)
            def _(c=c, g2=g2):
                s = (c + 2) % M
                pltpu.make_async_copy(idx_hbm.at[pl.ds(g2 * CH, CH)],
                                      idx[s], sem_i).wait()
                if c >= 2:
                    wait_write(c - 2, g2 - 4 * NW)
                pltpu.async_copy(wsum_hbm.at[idx[s]], rows[s], sem_g[s])
                if c + 3 < CPW:
                    @pl.when(g2 + NW < NCH)
                    def _():
                        pltpu.async_copy(
                            idx_hbm.at[pl.ds((g2 + NW) * CH, CH)],
                            idx[(c + 3) % M], sem_i)

        if c < CPW - 1:
            wait_gather(c)
            pltpu.async_copy(rows[b], out_hbm.at[pl.ds(g * CH, CH)],
                             sem_w[b])
        else:
            @pl.when(g < NCH)
            def _(c=c):
                wait_gather(c)

            @pl.when(g < NFULL)
            def _(g=g, b=b):
                pltpu.async_copy(rows[b], out_hbm.at[pl.ds(g * CH, CH)],
                                 sem_w[b])

            @pl.when(g == NFULL)
            def _(g=g, b=b):
                pltpu.sync_copy(rows[b].at[pl.ds(0, TAIL)],
                                out_hbm.at[pl.ds(g * CH, TAIL)])

    wait_write(CPW - 4, (CPW - 4) * NW + wid)
    wait_write(CPW - 3, (CPW - 3) * NW + wid)
    wait_write(CPW - 2, (CPW - 2) * NW + wid)
    g24 = (CPW - 1) * NW + wid

    @pl.when(g24 < NFULL)
    def _():
        wait_write(CPW - 1, g24)

    @pl.when(g24 >= NCH)
    def _():
        wait_write(CPW - 5, (CPW - 5) * NW + wid)


def kernel(in_degree, out_degree, W_in, W_out):
    pad = PAD_N - N_ROWS
    ind = jnp.pad(in_degree.astype(jnp.int32), (0, pad)).reshape(NCH, CH)
    outd = jnp.pad(out_degree.astype(jnp.int32), (0, pad)).reshape(NCH, CH)
    wsum, idxc = pl.pallas_call(
        _prep_body,
        out_shape=[
            jax.ShapeDtypeStruct((VOCAB, VOCAB, D), jnp.float32),
            jax.ShapeDtypeStruct((NCH, CH), jnp.int32),
        ],
    )(W_in, W_out, ind, outd)
    return _sc_gather(wsum.reshape(VOCAB * VOCAB, D), idxc.reshape(PAD_N))
